# Initial kernel scaffold; baseline (speedup 1.0000x reference)
import jax
import jax.numpy as jnp
from jax.experimental import pallas as pl


def kernel(x, s, t, W_lin, b_lin, W_attn):
    raise NotImplementedError("write your pallas kernel here")


# speedup vs baseline: 4.1876x; 4.1876x over previous
import functools

import jax
import jax.numpy as jnp
from jax import lax
from jax.experimental import pallas as pl
from jax.experimental.pallas import tpu as pltpu
from jax.experimental.pallas import tpu_sc as plsc

N = 10000
E = 320000
D = 128
DEXT = 144
ALPHA = 0.2

NC, NS = 2, 16
NW = NC * NS
C = 64
NCHT = E // C
KE = -(-NCHT // NW)
ZB = 64
NZB = N // ZB
ZREM = N - NZB * ZB
KZ = -(-NZB // NS)


def _dense_body(x_ref, w_ref, b_ref, a_ref, h_ref, fg_ref):
    h = lax.dot_general(x_ref[...], w_ref[...], (((1,), (1,)), ((), ())),
                        preferred_element_type=jnp.float32) + b_ref[...]
    h_ref[...] = h
    fg_ref[...] = lax.dot_general(a_ref[...], h, (((1,), (1,)), ((), ())),
                                  preferred_element_type=jnp.float32)


def _dense(x, W_lin, b_lin, a_mat):
    return pl.pallas_call(
        _dense_body,
        out_shape=[
            jax.ShapeDtypeStruct((N, D), jnp.float32),
            jax.ShapeDtypeStruct((2, N), jnp.float32),
        ],
    )(x, W_lin, b_lin, a_mat)


@functools.cache
def _make_sc_edge():
  mesh = plsc.VectorSubcoreMesh(core_axis_name="c", subcore_axis_name="s")

  @functools.partial(
      pl.kernel,
      mesh=mesh,
      compiler_params=pltpu.CompilerParams(
          needs_layout_passes=False, use_tc_tiling_on_sc=False),
      out_type=jax.ShapeDtypeStruct((NC, N, DEXT), jnp.float32),
      scratch_types=[
          pltpu.VMEM((C,), jnp.int32),
          pltpu.VMEM((C,), jnp.int32),
          pltpu.VMEM((C, D), jnp.float32),
          pltpu.VMEM((C, DEXT), jnp.float32),
          pltpu.VMEM((N,), jnp.float32),
          pltpu.VMEM((N,), jnp.float32),
          pltpu.VMEM((C,), jnp.float32),
          pltpu.VMEM_SHARED((N, DEXT), jnp.float32),
          pltpu.SemaphoreType.DMA,
      ],
  )
  def _sc_edge(h_hbm, fg_hbm, s_hbm, t_hbm, out_hbm,
               sidx, tidx, rows, scaled, ftab, gtab, wbuf, aggsh, sem):
    cid = lax.axis_index("c")
    sid = lax.axis_index("s")
    wid = cid * NS + sid

    pltpu.sync_copy(fg_hbm.at[pl.ds(0, N)], ftab)
    pltpu.sync_copy(fg_hbm.at[pl.ds(N, N)], gtab)

    zeros16 = jnp.zeros((16,), jnp.float32)

    @pl.loop(0, ZB)
    def _zero_scaled(i):
        for j in range(DEXT // 16):
            scaled[i, pl.ds(j * 16, 16)] = zeros16

    for k in range(KZ):
        blk = sid + NS * k

        @pl.when(blk < NZB)
        def _zero_agg():
            pltpu.sync_copy(
                scaled, aggsh.at[pl.ds(pl.multiple_of(blk * ZB, ZB), ZB)])

    @pl.when(sid == 0)
    def _zero_rem():
        pltpu.sync_copy(scaled.at[pl.ds(0, ZREM)],
                        aggsh.at[pl.ds(NZB * ZB, ZREM)])

    plsc.subcore_barrier()

    lane_is0 = lax.iota(jnp.int32, 16) == 0

    @pl.loop(0, KE)
    def _chunk(k):
        ch = wid + NW * k

        @pl.when(ch < NCHT)
        def _do_chunk():
            eb = pl.multiple_of(ch * C, C)
            pltpu.sync_copy(s_hbm.at[pl.ds(eb, C)], sidx)
            pltpu.sync_copy(t_hbm.at[pl.ds(eb, C)], tidx)
            pltpu.async_copy(h_hbm.at[tidx], rows, sem).wait()

            for grp in range(C // 16):
                sv = sidx[pl.ds(grp * 16, 16)]
                tv = tidx[pl.ds(grp * 16, 16)]
                e = plsc.load_gather(ftab, [sv]) + plsc.load_gather(gtab, [tv])
                e = jnp.where(e >= 0.0, e, ALPHA * e)
                wbuf[pl.ds(grp * 16, 16)] = jnp.exp(e)

            @pl.loop(0, C)
            def _scale(i):
                wv = plsc.load_gather(wbuf, [jnp.full((16,), i, jnp.int32)])
                for j in range(D // 16):
                    scaled[i, pl.ds(j * 16, 16)] = rows[i, pl.ds(j * 16, 16)] * wv
                scaled[i, pl.ds(D, 16)] = jnp.where(lane_is0, wv, 0.0)

            pltpu.sync_copy(scaled, aggsh.at[sidx], add=True)

    plsc.subcore_barrier()

    for k in range(KZ):
        blk = sid + NS * k

        @pl.when(blk < NZB)
        def _writeback():
            r0 = pl.multiple_of(blk * ZB, ZB)
            pltpu.sync_copy(aggsh.at[pl.ds(r0, ZB)], scaled)
            pltpu.sync_copy(scaled, out_hbm.at[cid, pl.ds(r0, ZB)])

    @pl.when(sid == NS - 1)
    def _writeback_rem():
        pltpu.sync_copy(aggsh.at[pl.ds(NZB * ZB, ZREM)],
                        scaled.at[pl.ds(0, ZREM)])
        pltpu.sync_copy(scaled.at[pl.ds(0, ZREM)],
                        out_hbm.at[cid, pl.ds(NZB * ZB, ZREM)])

  return _sc_edge


def _combine_body(a0_ref, a1_ref, o_ref):
    sm = a0_ref[...] + a1_ref[...]
    o_ref[...] = sm[:, :D] / sm[:, D:D + 1]


def _combine(a0, a1):
    B = 2000
    return pl.pallas_call(
        _combine_body,
        grid=(N // B,),
        in_specs=[
            pl.BlockSpec((B, DEXT), lambda i: (i, 0)),
            pl.BlockSpec((B, DEXT), lambda i: (i, 0)),
        ],
        out_specs=pl.BlockSpec((B, D), lambda i: (i, 0)),
        out_shape=jax.ShapeDtypeStruct((N, D), jnp.float32),
    )(a0, a1)


def kernel(x, s, t, W_lin, b_lin, W_attn):
    a_mat = W_attn.reshape(2, D)
    h, fg = _dense(x, W_lin, b_lin.reshape(1, D), a_mat)
    parts = _make_sc_edge()(h, fg.reshape(2 * N), s, t)
    return _combine(parts[0], parts[1])


# speedup vs baseline: 14.2109x; 3.3936x over previous
import functools

import jax
import jax.numpy as jnp
from jax import lax
from jax.experimental import pallas as pl
from jax.experimental.pallas import tpu as pltpu
from jax.experimental.pallas import tpu_sc as plsc

N = 10000
E = 320000
D = 128
ALPHA = 0.2

NC, NS = 2, 16
NW = NC * NS
C = 80
CPW = E // C // NW
ZBLK = 80
NZB = N // ZBLK
KZ = -(-NZB // NS)


def _dense_body(x_ref, w_ref, b_ref, a_ref, h_ref, fg_ref):
    h = lax.dot_general(x_ref[...], w_ref[...], (((1,), (1,)), ((), ())),
                        preferred_element_type=jnp.float32) + b_ref[...]
    h_ref[...] = h
    fg_ref[...] = lax.dot_general(a_ref[...], h, (((1,), (1,)), ((), ())),
                                  preferred_element_type=jnp.float32)


def _dense(x, W_lin, b_lin, a_mat):
    return pl.pallas_call(
        _dense_body,
        out_shape=[
            jax.ShapeDtypeStruct((N, D), jnp.float32),
            jax.ShapeDtypeStruct((2, N), jnp.float32),
        ],
    )(x, W_lin, b_lin, a_mat)


@functools.cache
def _make_sc_edge():
  mesh = plsc.VectorSubcoreMesh(core_axis_name="c", subcore_axis_name="s")

  @functools.partial(
      pl.kernel,
      mesh=mesh,
      compiler_params=pltpu.CompilerParams(
          needs_layout_passes=False, use_tc_tiling_on_sc=False),
      out_type=[
          jax.ShapeDtypeStruct((NC, N, D), jnp.float32),
          jax.ShapeDtypeStruct((NC, N), jnp.float32),
      ],
      scratch_types=[
          pltpu.VMEM((C,), jnp.int32),
          pltpu.VMEM((C,), jnp.int32),
          pltpu.VMEM((C,), jnp.int32),
          pltpu.VMEM((C,), jnp.int32),
          pltpu.VMEM((C,), jnp.int32),
          pltpu.VMEM((C,), jnp.int32),
          pltpu.VMEM((C, D), jnp.float32),
          pltpu.VMEM((C, D), jnp.float32),
          pltpu.VMEM((C,), jnp.float32),
          pltpu.VMEM((C,), jnp.float32),
          pltpu.VMEM((N,), jnp.float32),
          pltpu.VMEM((N,), jnp.float32),
          pltpu.VMEM_SHARED((N, D), jnp.float32),
          pltpu.VMEM_SHARED((N,), jnp.float32),
          pltpu.SemaphoreType.DMA,
          pltpu.SemaphoreType.DMA,
          pltpu.SemaphoreType.DMA,
          pltpu.SemaphoreType.DMA,
          pltpu.SemaphoreType.DMA,
          pltpu.SemaphoreType.DMA,
          pltpu.SemaphoreType.DMA,
          pltpu.SemaphoreType.DMA,
          pltpu.SemaphoreType.DMA,
          pltpu.SemaphoreType.DMA,
      ],
  )
  def _sc_edge(h_hbm, fg_hbm, s_hbm, t_hbm, agg_hbm, div_hbm,
               sidx0, sidx1, tidx0, tidx1, scat0, scat1, rows0, rows1,
               wbuf0, wbuf1, ftab, gtab, aggsh, divsh,
               ss0, ss1, st0, st1, sg0, sg1, sr0, sr1, sw0, sw1):
    cid = lax.axis_index("c")
    sid = lax.axis_index("s")
    wid = cid * NS + sid
    base = wid * CPW * C

    sidx = (sidx0, sidx1)
    tidx = (tidx0, tidx1)
    scat = (scat0, scat1)
    rows = (rows0, rows1)
    wbuf = (wbuf0, wbuf1)
    sem_s = (ss0, ss1)
    sem_t = (st0, st1)
    sem_g = (sg0, sg1)
    sem_r = (sr0, sr1)
    sem_w = (sw0, sw1)

    zeros16 = jnp.zeros((16,), jnp.float32)

    @pl.loop(0, N // 16)
    def _zero_ftab(i):
        ftab[pl.ds(pl.multiple_of(i * 16, 16), 16)] = zeros16

    @pl.when(sid == 0)
    def _zero_div():
        pltpu.sync_copy(ftab, divsh)

    @pl.loop(0, C)
    def _zero_rows0(i):
        for j in range(D // 16):
            rows0[i, pl.ds(j * 16, 16)] = zeros16

    for k in range(KZ):
        blk = sid + NS * k

        @pl.when(blk < NZB)
        def _zero_agg():
            pltpu.sync_copy(
                rows0, aggsh.at[pl.ds(pl.multiple_of(blk * ZBLK, ZBLK), ZBLK)])

    pltpu.sync_copy(fg_hbm.at[pl.ds(0, N)], ftab)
    pltpu.sync_copy(fg_hbm.at[pl.ds(N, N)], gtab)

    plsc.subcore_barrier()

    def start_idx(k, p):
        eb = pl.multiple_of(base + k * C, C)
        pltpu.async_copy(s_hbm.at[pl.ds(eb, C)], sidx[p], sem_s[p])
        pltpu.async_copy(t_hbm.at[pl.ds(eb, C)], tidx[p], sem_t[p])

    def wait_idx(k, p):
        eb = pl.multiple_of(base + k * C, C)
        pltpu.make_async_copy(s_hbm.at[pl.ds(eb, C)], sidx[p], sem_s[p]).wait()
        pltpu.make_async_copy(t_hbm.at[pl.ds(eb, C)], tidx[p], sem_t[p]).wait()

    def start_gather(p):
        pltpu.async_copy(h_hbm.at[tidx[p]], rows[p], sem_g[p])

    def step(k, p, q):
        static = isinstance(k, int)
        pltpu.make_async_copy(h_hbm.at[tidx[p]], rows[p], sem_g[p]).wait()

        def prefetch_gather():
            wait_idx(k + 1, q)
            if (not static) or k >= 1:
                pltpu.make_async_copy(
                    rows[q], aggsh.at[scat[q]], sem_r[q]).wait()
            start_gather(q)

        if static:
            if k + 1 < CPW:
                prefetch_gather()
        else:
            prefetch_gather()

        if (not static) or k >= 2:
            pltpu.make_async_copy(
                wbuf[p], divsh.at[scat[p]], sem_w[p]).wait()
        for grp in range(C // 16):
            off = grp * 16
            sv = sidx[p][pl.ds(off, 16)]
            tv = tidx[p][pl.ds(off, 16)]
            e = plsc.load_gather(ftab, [sv]) + plsc.load_gather(gtab, [tv])
            e = jnp.where(e >= 0.0, e, ALPHA * e)
            wbuf[p][pl.ds(off, 16)] = jnp.exp(e)
            scat[p][pl.ds(off, 16)] = sv

        @pl.loop(0, C)
        def _scale(i):
            wv = plsc.load_gather(wbuf[p], [jnp.full((16,), i, jnp.int32)])
            for j in range(D // 16):
                rows[p][i, pl.ds(j * 16, 16)] = (
                    rows[p][i, pl.ds(j * 16, 16)] * wv)

        pltpu.async_copy(rows[p], aggsh.at[scat[p]], sem_r[p], add=True)
        pltpu.async_copy(wbuf[p], divsh.at[scat[p]], sem_w[p], add=True)

        if static:
            if k + 2 < CPW:
                start_idx(k + 2, p)
        else:
            @pl.when(k + 2 < CPW)
            def _():
                start_idx(k + 2, p)

    start_idx(0, 0)
    start_idx(1, 1)
    wait_idx(0, 0)
    start_gather(0)
    step(0, 0, 1)
    step(1, 1, 0)

    @pl.loop(0, (CPW - 3) // 2)
    def _main(j):
        k = 2 + 2 * j
        step(k, 0, 1)
        step(k + 1, 1, 0)

    step(CPW - 1, 0, 1)

    pltpu.make_async_copy(rows[1], aggsh.at[scat[1]], sem_r[1]).wait()
    pltpu.make_async_copy(wbuf[1], divsh.at[scat[1]], sem_w[1]).wait()
    pltpu.make_async_copy(rows[0], aggsh.at[scat[0]], sem_r[0]).wait()
    pltpu.make_async_copy(wbuf[0], divsh.at[scat[0]], sem_w[0]).wait()

    plsc.subcore_barrier()

    for k in range(KZ):
        blk = sid + NS * k

        @pl.when(blk < NZB)
        def _writeback():
            r0 = pl.multiple_of(blk * ZBLK, ZBLK)
            pltpu.sync_copy(aggsh.at[pl.ds(r0, ZBLK)], rows0)
            pltpu.sync_copy(rows0, agg_hbm.at[cid, pl.ds(r0, ZBLK)])

    @pl.when(sid == 0)
    def _writeback_div():
        pltpu.sync_copy(divsh, ftab)
        pltpu.sync_copy(ftab, div_hbm.at[cid])

  return _sc_edge


def _combine_body(a0_ref, a1_ref, d0_ref, d1_ref, o_ref):
    o_ref[...] = (a0_ref[...] + a1_ref[...]) / (d0_ref[...] + d1_ref[...])


def _combine(a0, a1, d0, d1):
    B = 2000
    return pl.pallas_call(
        _combine_body,
        grid=(N // B,),
        in_specs=[
            pl.BlockSpec((B, D), lambda i: (i, 0)),
            pl.BlockSpec((B, D), lambda i: (i, 0)),
            pl.BlockSpec((B, 1), lambda i: (i, 0)),
            pl.BlockSpec((B, 1), lambda i: (i, 0)),
        ],
        out_specs=pl.BlockSpec((B, D), lambda i: (i, 0)),
        out_shape=jax.ShapeDtypeStruct((N, D), jnp.float32),
    )(a0, a1, d0, d1)


def kernel(x, s, t, W_lin, b_lin, W_attn):
    a_mat = W_attn.reshape(2, D)
    h, fg = _dense(x, W_lin, b_lin.reshape(1, D), a_mat)
    aggs, divs = _make_sc_edge()(h, fg.reshape(2 * N), s, t)
    return _combine(aggs[0], aggs[1],
                    divs[0].reshape(N, 1), divs[1].reshape(N, 1))


# speedup vs baseline: 16.1008x; 1.1330x over previous
import functools

import jax
import jax.numpy as jnp
from jax import lax
from jax.experimental import pallas as pl
from jax.experimental.pallas import tpu as pltpu
from jax.experimental.pallas import tpu_sc as plsc

N = 10000
E = 320000
D = 128
ALPHA = 0.2

NC, NS = 2, 16
NW = NC * NS
C = 80
CPW = E // C // NW
ZBLK = 80
NZB = N // ZBLK
KZ = -(-NZB // NS)


def _dense_body(x_ref, w_ref, b_ref, a_ref, h_ref, fg_ref):
    h = lax.dot_general(x_ref[...], w_ref[...], (((1,), (1,)), ((), ())),
                        preferred_element_type=jnp.float32) + b_ref[...]
    h_ref[...] = h
    fg_ref[...] = lax.dot_general(a_ref[...], h, (((1,), (1,)), ((), ())),
                                  preferred_element_type=jnp.float32)


def _dense(x, W_lin, b_lin, a_mat):
    return pl.pallas_call(
        _dense_body,
        out_shape=[
            jax.ShapeDtypeStruct((N, D), jnp.float32),
            jax.ShapeDtypeStruct((2, N), jnp.float32),
        ],
    )(x, W_lin, b_lin, a_mat)


@functools.cache
def _make_sc_edge():
  mesh = plsc.VectorSubcoreMesh(core_axis_name="c", subcore_axis_name="s")

  @functools.partial(
      pl.kernel,
      mesh=mesh,
      compiler_params=pltpu.CompilerParams(
          needs_layout_passes=False, use_tc_tiling_on_sc=False),
      out_type=[
          jax.ShapeDtypeStruct((NC, N, D), jnp.float32),
          jax.ShapeDtypeStruct((NC, N), jnp.float32),
      ],
      scratch_types=[
          pltpu.VMEM((C,), jnp.int32),
          pltpu.VMEM((C,), jnp.int32),
          pltpu.VMEM((C,), jnp.int32),
          pltpu.VMEM((C,), jnp.int32),
          pltpu.VMEM((C,), jnp.int32),
          pltpu.VMEM((C,), jnp.int32),
          pltpu.VMEM((C, D), jnp.float32),
          pltpu.VMEM((C, D), jnp.float32),
          pltpu.VMEM((C,), jnp.float32),
          pltpu.VMEM((C,), jnp.float32),
          pltpu.VMEM((N,), jnp.float32),
          pltpu.VMEM((N,), jnp.float32),
          pltpu.VMEM_SHARED((N, D), jnp.float32),
          pltpu.VMEM_SHARED((N,), jnp.float32),
          pltpu.SemaphoreType.DMA,
          pltpu.SemaphoreType.DMA,
          pltpu.SemaphoreType.DMA,
          pltpu.SemaphoreType.DMA,
          pltpu.SemaphoreType.DMA,
          pltpu.SemaphoreType.DMA,
          pltpu.SemaphoreType.DMA,
          pltpu.SemaphoreType.DMA,
          pltpu.SemaphoreType.DMA,
          pltpu.SemaphoreType.DMA,
      ],
  )
  def _sc_edge(h_hbm, fg_hbm, s_hbm, t_hbm, agg_hbm, div_hbm,
               sidx0, sidx1, tidx0, tidx1, scat0, scat1, rows0, rows1,
               wbuf0, wbuf1, ftab, gtab, aggsh, divsh,
               ss0, ss1, st0, st1, sg0, sg1, sr0, sr1, sw0, sw1):
    cid = lax.axis_index("c")
    sid = lax.axis_index("s")
    wid = cid * NS + sid
    base = wid * CPW * C

    sidx = (sidx0, sidx1)
    tidx = (tidx0, tidx1)
    scat = (scat0, scat1)
    rows = (rows0, rows1)
    wbuf = (wbuf0, wbuf1)
    sem_s = (ss0, ss1)
    sem_t = (st0, st1)
    sem_g = (sg0, sg1)
    sem_r = (sr0, sr1)
    sem_w = (sw0, sw1)

    zeros16 = jnp.zeros((16,), jnp.float32)

    @pl.loop(0, N // 16)
    def _zero_ftab(i):
        ftab[pl.ds(pl.multiple_of(i * 16, 16), 16)] = zeros16

    @pl.when(sid == 0)
    def _zero_div():
        pltpu.sync_copy(ftab, divsh)

    @pl.loop(0, C)
    def _zero_rows0(i):
        for j in range(D // 16):
            rows0[i, pl.ds(j * 16, 16)] = zeros16

    for k in range(KZ):
        blk = sid + NS * k

        @pl.when(blk < NZB)
        def _zero_agg():
            pltpu.sync_copy(
                rows0, aggsh.at[pl.ds(pl.multiple_of(blk * ZBLK, ZBLK), ZBLK)])

    pltpu.sync_copy(fg_hbm.at[pl.ds(0, N)], ftab)
    pltpu.sync_copy(fg_hbm.at[pl.ds(N, N)], gtab)

    plsc.subcore_barrier()

    def start_idx(k, p):
        eb = pl.multiple_of(base + k * C, C)
        pltpu.async_copy(s_hbm.at[pl.ds(eb, C)], sidx[p], sem_s[p])
        pltpu.async_copy(t_hbm.at[pl.ds(eb, C)], tidx[p], sem_t[p])

    def wait_idx(k, p):
        eb = pl.multiple_of(base + k * C, C)
        pltpu.make_async_copy(s_hbm.at[pl.ds(eb, C)], sidx[p], sem_s[p]).wait()
        pltpu.make_async_copy(t_hbm.at[pl.ds(eb, C)], tidx[p], sem_t[p]).wait()

    def start_gather(p):
        pltpu.async_copy(h_hbm.at[tidx[p]], rows[p], sem_g[p])

    def step(k, p, q):
        static = isinstance(k, int)
        pltpu.make_async_copy(h_hbm.at[tidx[p]], rows[p], sem_g[p]).wait()

        def prefetch_gather():
            wait_idx(k + 1, q)
            if (not static) or k >= 1:
                pltpu.make_async_copy(
                    rows[q], aggsh.at[scat[q]], sem_r[q]).wait()
            start_gather(q)

        if static:
            if k + 1 < CPW:
                prefetch_gather()
        else:
            prefetch_gather()

        if (not static) or k >= 2:
            pltpu.make_async_copy(
                wbuf[p], divsh.at[scat[p]], sem_w[p]).wait()
        for grp in range(C // 16):
            off = grp * 16
            sv = sidx[p][pl.ds(off, 16)]
            tv = tidx[p][pl.ds(off, 16)]
            e = plsc.load_gather(ftab, [sv]) + plsc.load_gather(gtab, [tv])
            e = jnp.where(e >= 0.0, e, ALPHA * e)
            wbuf[p][pl.ds(off, 16)] = jnp.exp(e)
            scat[p][pl.ds(off, 16)] = sv

        @pl.loop(0, C // 2)
        def _scale(ih):
            i = ih * 2
            wv0 = plsc.load_gather(wbuf[p], [jnp.full((16,), i, jnp.int32)])
            wv1 = plsc.load_gather(
                wbuf[p], [jnp.full((16,), i + 1, jnp.int32)])
            for j in range(D // 16):
                rows[p][i, pl.ds(j * 16, 16)] = (
                    rows[p][i, pl.ds(j * 16, 16)] * wv0)
            for j in range(D // 16):
                rows[p][i + 1, pl.ds(j * 16, 16)] = (
                    rows[p][i + 1, pl.ds(j * 16, 16)] * wv1)

        pltpu.async_copy(rows[p], aggsh.at[scat[p]], sem_r[p], add=True)
        pltpu.async_copy(wbuf[p], divsh.at[scat[p]], sem_w[p], add=True)

        if static:
            if k + 2 < CPW:
                start_idx(k + 2, p)
        else:
            @pl.when(k + 2 < CPW)
            def _():
                start_idx(k + 2, p)

    start_idx(0, 0)
    start_idx(1, 1)
    wait_idx(0, 0)
    start_gather(0)
    step(0, 0, 1)
    step(1, 1, 0)

    @pl.loop(0, (CPW - 3) // 2)
    def _main(j):
        k = 2 + 2 * j
        step(k, 0, 1)
        step(k + 1, 1, 0)

    step(CPW - 1, 0, 1)

    pltpu.make_async_copy(rows[1], aggsh.at[scat[1]], sem_r[1]).wait()
    pltpu.make_async_copy(wbuf[1], divsh.at[scat[1]], sem_w[1]).wait()
    pltpu.make_async_copy(rows[0], aggsh.at[scat[0]], sem_r[0]).wait()
    pltpu.make_async_copy(wbuf[0], divsh.at[scat[0]], sem_w[0]).wait()

    plsc.subcore_barrier()

    for k in range(KZ):
        blk = sid + NS * k

        @pl.when(blk < NZB)
        def _writeback():
            r0 = pl.multiple_of(blk * ZBLK, ZBLK)
            pltpu.sync_copy(aggsh.at[pl.ds(r0, ZBLK)],
                            agg_hbm.at[cid, pl.ds(r0, ZBLK)])

    @pl.when(sid == 0)
    def _writeback_div():
        pltpu.sync_copy(divsh, div_hbm.at[cid])

  return _sc_edge


def _combine_body(a0_ref, a1_ref, d0_ref, d1_ref, o_ref):
    o_ref[...] = (a0_ref[...] + a1_ref[...]) / (d0_ref[...] + d1_ref[...])


def _combine(a0, a1, d0, d1):
    B = 2000
    return pl.pallas_call(
        _combine_body,
        grid=(N // B,),
        in_specs=[
            pl.BlockSpec((B, D), lambda i: (i, 0)),
            pl.BlockSpec((B, D), lambda i: (i, 0)),
            pl.BlockSpec((B, 1), lambda i: (i, 0)),
            pl.BlockSpec((B, 1), lambda i: (i, 0)),
        ],
        out_specs=pl.BlockSpec((B, D), lambda i: (i, 0)),
        out_shape=jax.ShapeDtypeStruct((N, D), jnp.float32),
    )(a0, a1, d0, d1)


def kernel(x, s, t, W_lin, b_lin, W_attn):
    a_mat = W_attn.reshape(2, D)
    h, fg = _dense(x, W_lin, b_lin.reshape(1, D), a_mat)
    aggs, divs = _make_sc_edge()(h, fg.reshape(2 * N), s, t)
    return _combine(aggs[0], aggs[1],
                    divs[0].reshape(N, 1), divs[1].reshape(N, 1))


# speedup vs baseline: 16.1280x; 1.0017x over previous
import functools

import jax
import jax.numpy as jnp
from jax import lax
from jax.experimental import pallas as pl
from jax.experimental.pallas import tpu as pltpu
from jax.experimental.pallas import tpu_sc as plsc

N = 10000
E = 320000
D = 128
ALPHA = 0.2

NC, NS = 2, 16
NW = NC * NS
C = 80
CPW = E // C // NW
SEG = 25
NSEG = CPW // SEG
ZBLK = 80
NZB = N // ZBLK
KZ = -(-NZB // NS)


def _dense_body(x_ref, w_ref, b_ref, a_ref, h_ref, fg_ref):
    h = lax.dot_general(x_ref[...], w_ref[...], (((1,), (1,)), ((), ())),
                        preferred_element_type=jnp.float32) + b_ref[...]
    h_ref[...] = h
    fg_ref[...] = lax.dot_general(a_ref[...], h, (((1,), (1,)), ((), ())),
                                  preferred_element_type=jnp.float32)


def _dense(x, W_lin, b_lin, a_mat):
    return pl.pallas_call(
        _dense_body,
        out_shape=[
            jax.ShapeDtypeStruct((N, D), jnp.float32),
            jax.ShapeDtypeStruct((2, N), jnp.float32),
        ],
    )(x, W_lin, b_lin, a_mat)


@functools.cache
def _make_sc_edge():
  mesh = plsc.VectorSubcoreMesh(core_axis_name="c", subcore_axis_name="s")

  @functools.partial(
      pl.kernel,
      mesh=mesh,
      compiler_params=pltpu.CompilerParams(
          needs_layout_passes=False, use_tc_tiling_on_sc=False),
      out_type=[
          jax.ShapeDtypeStruct((NC, N, D), jnp.float32),
          jax.ShapeDtypeStruct((NC, N), jnp.float32),
      ],
      scratch_types=[
          pltpu.VMEM((SEG * C,), jnp.int32),
          pltpu.VMEM((SEG * C,), jnp.int32),
          pltpu.VMEM((SEG * C,), jnp.int32),
          pltpu.VMEM((SEG * C,), jnp.int32),
          pltpu.VMEM((C,), jnp.int32),
          pltpu.VMEM((C,), jnp.int32),
          pltpu.VMEM((C, D), jnp.float32),
          pltpu.VMEM((C, D), jnp.float32),
          pltpu.VMEM((C,), jnp.float32),
          pltpu.VMEM((C,), jnp.float32),
          pltpu.VMEM((N,), jnp.float32),
          pltpu.VMEM((N,), jnp.float32),
          pltpu.VMEM_SHARED((N, D), jnp.float32),
          pltpu.VMEM_SHARED((N,), jnp.float32),
          pltpu.SemaphoreType.DMA,
          pltpu.SemaphoreType.DMA,
          pltpu.SemaphoreType.DMA,
          pltpu.SemaphoreType.DMA,
          pltpu.SemaphoreType.DMA,
          pltpu.SemaphoreType.DMA,
          pltpu.SemaphoreType.DMA,
          pltpu.SemaphoreType.DMA,
          pltpu.SemaphoreType.DMA,
          pltpu.SemaphoreType.DMA,
      ],
  )
  def _sc_edge(h_hbm, fg_hbm, s_hbm, t_hbm, agg_hbm, div_hbm,
               sbig0, sbig1, tbig0, tbig1, scat0, scat1, rows0, rows1,
               wbuf0, wbuf1, ftab, gtab, aggsh, divsh,
               ss0, ss1, st0, st1, sg0, sg1, sr0, sr1, sw0, sw1):
    cid = lax.axis_index("c")
    sid = lax.axis_index("s")
    wid = cid * NS + sid
    base = wid * CPW * C

    sbig = (sbig0, sbig1)
    tbig = (tbig0, tbig1)
    scat = (scat0, scat1)
    rows = (rows0, rows1)
    wbuf = (wbuf0, wbuf1)
    sem_s = (ss0, ss1)
    sem_t = (st0, st1)
    sem_g = (sg0, sg1)
    sem_r = (sr0, sr1)
    sem_w = (sw0, sw1)

    zeros16 = jnp.zeros((16,), jnp.float32)

    @pl.loop(0, N // 16)
    def _zero_ftab(i):
        ftab[pl.ds(pl.multiple_of(i * 16, 16), 16)] = zeros16

    @pl.when(sid == 0)
    def _zero_div():
        pltpu.sync_copy(ftab, divsh)

    @pl.loop(0, C)
    def _zero_rows0(i):
        for j in range(D // 16):
            rows0[i, pl.ds(j * 16, 16)] = zeros16

    for k in range(KZ):
        blk = sid + NS * k

        @pl.when(blk < NZB)
        def _zero_agg():
            pltpu.sync_copy(
                rows0, aggsh.at[pl.ds(pl.multiple_of(blk * ZBLK, ZBLK), ZBLK)])

    pltpu.sync_copy(fg_hbm.at[pl.ds(0, N)], ftab)
    pltpu.sync_copy(fg_hbm.at[pl.ds(N, N)], gtab)

    plsc.subcore_barrier()

    def start_seg(g):
        eb = pl.multiple_of(base + g * SEG * C, C)
        m = g % 2
        pltpu.async_copy(s_hbm.at[pl.ds(eb, SEG * C)], sbig[m], sem_s[m])
        pltpu.async_copy(t_hbm.at[pl.ds(eb, SEG * C)], tbig[m], sem_t[m])

    def wait_seg(g):
        eb = pl.multiple_of(base + g * SEG * C, C)
        m = g % 2
        pltpu.make_async_copy(
            s_hbm.at[pl.ds(eb, SEG * C)], sbig[m], sem_s[m]).wait()
        pltpu.make_async_copy(
            t_hbm.at[pl.ds(eb, SEG * C)], tbig[m], sem_t[m]).wait()

    def gidx(tb, lc):
        return tb.at[pl.ds(pl.multiple_of(lc * C, C), C)]

    def start_gather(tb, lc, q):
        pltpu.async_copy(h_hbm.at[gidx(tb, lc)], rows[q], sem_g[q])

    def step(p, q, sb, tb, lc, tb1=None, lc1=None,
             first_r=False, first_w=False):
        pltpu.make_async_copy(h_hbm.at[gidx(tb, lc)], rows[p], sem_g[p]).wait()

        if tb1 is not None:
            if not first_r:
                pltpu.make_async_copy(
                    rows[q], aggsh.at[scat[q]], sem_r[q]).wait()
            start_gather(tb1, lc1, q)

        if not first_w:
            pltpu.make_async_copy(
                wbuf[p], divsh.at[scat[p]], sem_w[p]).wait()
        cbase = lc * C
        for grp in range(C // 16):
            off = pl.multiple_of(cbase + grp * 16, 16)
            sv = sb[pl.ds(off, 16)]
            tv = tb[pl.ds(off, 16)]
            e = plsc.load_gather(ftab, [sv]) + plsc.load_gather(gtab, [tv])
            e = jnp.where(e >= 0.0, e, ALPHA * e)
            wbuf[p][pl.ds(grp * 16, 16)] = jnp.exp(e)
            scat[p][pl.ds(grp * 16, 16)] = sv

        @pl.loop(0, C // 2)
        def _scale(ih):
            i = ih * 2
            wv0 = plsc.load_gather(wbuf[p], [jnp.full((16,), i, jnp.int32)])
            wv1 = plsc.load_gather(
                wbuf[p], [jnp.full((16,), i + 1, jnp.int32)])
            for j in range(D // 16):
                rows[p][i, pl.ds(j * 16, 16)] = (
                    rows[p][i, pl.ds(j * 16, 16)] * wv0)
            for j in range(D // 16):
                rows[p][i + 1, pl.ds(j * 16, 16)] = (
                    rows[p][i + 1, pl.ds(j * 16, 16)] * wv1)

        pltpu.async_copy(rows[p], aggsh.at[scat[p]], sem_r[p], add=True)
        pltpu.async_copy(wbuf[p], divsh.at[scat[p]], sem_w[p], add=True)

    start_seg(0)
    wait_seg(0)
    start_gather(tbig[0], 0, 0)

    for seg in range(NSEG):
        sb, tb = sbig[seg % 2], tbig[seg % 2]
        nxt = seg + 1 < NSEG
        tbn = tbig[(seg + 1) % 2] if nxt else None
        if nxt:
            start_seg(seg + 1)
        par = (SEG * seg) % 2

        if seg == 0:
            step(0, 1, sb, tb, 0, tb, 1, first_r=True, first_w=True)
            step(1, 0, sb, tb, 1, tb, 2, first_w=True)
            body_lo, body_pairs = 2, (SEG - 1 - 2) // 2
        else:
            body_lo, body_pairs = 0, (SEG - 1) // 2

        @pl.loop(0, body_pairs)
        def _pairs(j):
            c = body_lo + 2 * j
            step(par, 1 - par, sb, tb, c, tb, c + 1)
            step(1 - par, par, sb, tb, c + 1, tb, c + 2)

        lpar = (SEG * seg + SEG - 1) % 2
        if nxt:
            wait_seg(seg + 1)
            step(lpar, 1 - lpar, sb, tb, SEG - 1, tbn, 0)
        else:
            step(lpar, 1 - lpar, sb, tb, SEG - 1)

    pltpu.make_async_copy(rows[1], aggsh.at[scat[1]], sem_r[1]).wait()
    pltpu.make_async_copy(wbuf[1], divsh.at[scat[1]], sem_w[1]).wait()
    pltpu.make_async_copy(rows[0], aggsh.at[scat[0]], sem_r[0]).wait()
    pltpu.make_async_copy(wbuf[0], divsh.at[scat[0]], sem_w[0]).wait()

    plsc.subcore_barrier()

    for k in range(KZ):
        blk = sid + NS * k

        @pl.when(blk < NZB)
        def _writeback():
            r0 = pl.multiple_of(blk * ZBLK, ZBLK)
            pltpu.sync_copy(aggsh.at[pl.ds(r0, ZBLK)],
                            agg_hbm.at[cid, pl.ds(r0, ZBLK)])

    @pl.when(sid == 0)
    def _writeback_div():
        pltpu.sync_copy(divsh, div_hbm.at[cid])

  return _sc_edge


def _combine_body(a0_ref, a1_ref, d0_ref, d1_ref, o_ref):
    o_ref[...] = (a0_ref[...] + a1_ref[...]) / (d0_ref[...] + d1_ref[...])


def _combine(a0, a1, d0, d1):
    B = 2000
    return pl.pallas_call(
        _combine_body,
        grid=(N // B,),
        in_specs=[
            pl.BlockSpec((B, D), lambda i: (i, 0)),
            pl.BlockSpec((B, D), lambda i: (i, 0)),
            pl.BlockSpec((B, 1), lambda i: (i, 0)),
            pl.BlockSpec((B, 1), lambda i: (i, 0)),
        ],
        out_specs=pl.BlockSpec((B, D), lambda i: (i, 0)),
        out_shape=jax.ShapeDtypeStruct((N, D), jnp.float32),
    )(a0, a1, d0, d1)


def kernel(x, s, t, W_lin, b_lin, W_attn):
    a_mat = W_attn.reshape(2, D)
    h, fg = _dense(x, W_lin, b_lin.reshape(1, D), a_mat)
    aggs, divs = _make_sc_edge()(h, fg.reshape(2 * N), s, t)
    return _combine(aggs[0], aggs[1],
                    divs[0].reshape(N, 1), divs[1].reshape(N, 1))


# speedup vs baseline: 16.1428x; 1.0009x over previous
import functools

import jax
import jax.numpy as jnp
from jax import lax
from jax.experimental import pallas as pl
from jax.experimental.pallas import tpu as pltpu
from jax.experimental.pallas import tpu_sc as plsc

N = 10000
E = 320000
D = 128
ALPHA = 0.2

NC, NS = 2, 16
NW = NC * NS
C = 80
CPW = E // C // NW
SEG = 25
NSEG = CPW // SEG
ZBLK = 80
NZB = N // ZBLK
KZ = -(-NZB // NS)


def _dense_body(x_ref, w_ref, b_ref, a_ref, h_ref, fg_ref):
    h = lax.dot_general(x_ref[...], w_ref[...], (((1,), (1,)), ((), ())),
                        preferred_element_type=jnp.float32) + b_ref[...]
    h_ref[...] = h
    fg_ref[...] = lax.dot_general(a_ref[...], h, (((1,), (1,)), ((), ())),
                                  preferred_element_type=jnp.float32)


def _dense(x, W_lin, b_lin, a_mat):
    return pl.pallas_call(
        _dense_body,
        out_shape=[
            jax.ShapeDtypeStruct((N, D), jnp.float32),
            jax.ShapeDtypeStruct((2, N), jnp.float32),
        ],
    )(x, W_lin, b_lin, a_mat)


@functools.cache
def _make_sc_edge():
  mesh = plsc.VectorSubcoreMesh(core_axis_name="c", subcore_axis_name="s")

  @functools.partial(
      pl.kernel,
      mesh=mesh,
      compiler_params=pltpu.CompilerParams(
          needs_layout_passes=False, use_tc_tiling_on_sc=False),
      out_type=[
          jax.ShapeDtypeStruct((NC, N, D), jnp.float32),
          jax.ShapeDtypeStruct((NC, N), jnp.float32),
      ],
      scratch_types=[
          pltpu.VMEM((SEG * C,), jnp.int32),
          pltpu.VMEM((SEG * C,), jnp.int32),
          pltpu.VMEM((SEG * C,), jnp.int32),
          pltpu.VMEM((SEG * C,), jnp.int32),
          pltpu.VMEM((C,), jnp.int32),
          pltpu.VMEM((C,), jnp.int32),
          pltpu.VMEM((C, D), jnp.float32),
          pltpu.VMEM((C, D), jnp.float32),
          pltpu.VMEM((C,), jnp.float32),
          pltpu.VMEM((C,), jnp.float32),
          pltpu.VMEM((N,), jnp.float32),
          pltpu.VMEM((N,), jnp.float32),
          pltpu.VMEM_SHARED((N, D), jnp.float32),
          pltpu.VMEM_SHARED((N,), jnp.float32),
          pltpu.SemaphoreType.DMA,
          pltpu.SemaphoreType.DMA,
          pltpu.SemaphoreType.DMA,
          pltpu.SemaphoreType.DMA,
          pltpu.SemaphoreType.DMA,
          pltpu.SemaphoreType.DMA,
          pltpu.SemaphoreType.DMA,
          pltpu.SemaphoreType.DMA,
          pltpu.SemaphoreType.DMA,
          pltpu.SemaphoreType.DMA,
      ],
  )
  def _sc_edge(h_hbm, fg_hbm, s_hbm, t_hbm, agg_hbm, div_hbm,
               sbig0, sbig1, tbig0, tbig1, scat0, scat1, rows0, rows1,
               wbuf0, wbuf1, ftab, gtab, aggsh, divsh,
               ss0, ss1, st0, st1, sg0, sg1, sr0, sr1, sw0, sw1):
    cid = lax.axis_index("c")
    sid = lax.axis_index("s")
    wid = cid * NS + sid
    base = wid * CPW * C

    sbig = (sbig0, sbig1)
    tbig = (tbig0, tbig1)
    scat = (scat0, scat1)
    rows = (rows0, rows1)
    wbuf = (wbuf0, wbuf1)
    sem_s = (ss0, ss1)
    sem_t = (st0, st1)
    sem_g = (sg0, sg1)
    sem_r = (sr0, sr1)
    sem_w = (sw0, sw1)

    zeros16 = jnp.zeros((16,), jnp.float32)

    @pl.loop(0, N // 16)
    def _zero_ftab(i):
        ftab[pl.ds(pl.multiple_of(i * 16, 16), 16)] = zeros16

    @pl.when(sid == 0)
    def _zero_div():
        pltpu.sync_copy(ftab, divsh)

    @pl.loop(0, C)
    def _zero_rows0(i):
        for j in range(D // 16):
            rows0[i, pl.ds(j * 16, 16)] = zeros16

    for k in range(KZ):
        blk = sid + NS * k

        @pl.when(blk < NZB)
        def _zero_agg():
            pltpu.sync_copy(
                rows0, aggsh.at[pl.ds(pl.multiple_of(blk * ZBLK, ZBLK), ZBLK)])

    pltpu.sync_copy(fg_hbm.at[pl.ds(0, N)], ftab)
    pltpu.sync_copy(fg_hbm.at[pl.ds(N, N)], gtab)

    plsc.subcore_barrier()

    def start_seg(g):
        eb = pl.multiple_of(base + g * SEG * C, C)
        m = g % 2
        pltpu.async_copy(s_hbm.at[pl.ds(eb, SEG * C)], sbig[m], sem_s[m])
        pltpu.async_copy(t_hbm.at[pl.ds(eb, SEG * C)], tbig[m], sem_t[m])

    def wait_seg(g):
        eb = pl.multiple_of(base + g * SEG * C, C)
        m = g % 2
        pltpu.make_async_copy(
            s_hbm.at[pl.ds(eb, SEG * C)], sbig[m], sem_s[m]).wait()
        pltpu.make_async_copy(
            t_hbm.at[pl.ds(eb, SEG * C)], tbig[m], sem_t[m]).wait()

    H = C // 2

    def gidx(tb, lc, half):
        return tb.at[pl.ds(pl.multiple_of(lc * C + half * H, H), H)]

    def start_gather(tb, lc, q):
        pltpu.async_copy(
            h_hbm.at[gidx(tb, lc, 0)], rows[q].at[pl.ds(0, H)], sem_g[q])
        pltpu.async_copy(
            h_hbm.at[gidx(tb, lc, 1)], rows[q].at[pl.ds(H, H)], sem_g[q])

    def step(p, q, sb, tb, lc, tb1=None, lc1=None,
             first_r=False, first_w=False):
        pltpu.make_async_copy(
            h_hbm.at[gidx(tb, lc, 0)], rows[p].at[pl.ds(0, H)], sem_g[p]).wait()
        pltpu.make_async_copy(
            h_hbm.at[gidx(tb, lc, 1)], rows[p].at[pl.ds(H, H)], sem_g[p]).wait()

        if tb1 is not None:
            if not first_r:
                pltpu.make_async_copy(
                    rows[q], aggsh.at[scat[q]], sem_r[q]).wait()
            start_gather(tb1, lc1, q)

        if not first_w:
            pltpu.make_async_copy(
                wbuf[p], divsh.at[scat[p]], sem_w[p]).wait()
        cbase = lc * C
        for grp in range(C // 16):
            off = pl.multiple_of(cbase + grp * 16, 16)
            sv = sb[pl.ds(off, 16)]
            tv = tb[pl.ds(off, 16)]
            e = plsc.load_gather(ftab, [sv]) + plsc.load_gather(gtab, [tv])
            e = jnp.where(e >= 0.0, e, ALPHA * e)
            wbuf[p][pl.ds(grp * 16, 16)] = jnp.exp(e)
            scat[p][pl.ds(grp * 16, 16)] = sv

        @pl.loop(0, C // 2)
        def _scale(ih):
            i = ih * 2
            wv0 = plsc.load_gather(wbuf[p], [jnp.full((16,), i, jnp.int32)])
            wv1 = plsc.load_gather(
                wbuf[p], [jnp.full((16,), i + 1, jnp.int32)])
            for j in range(D // 16):
                rows[p][i, pl.ds(j * 16, 16)] = (
                    rows[p][i, pl.ds(j * 16, 16)] * wv0)
            for j in range(D // 16):
                rows[p][i + 1, pl.ds(j * 16, 16)] = (
                    rows[p][i + 1, pl.ds(j * 16, 16)] * wv1)

        pltpu.async_copy(rows[p], aggsh.at[scat[p]], sem_r[p], add=True)
        pltpu.async_copy(wbuf[p], divsh.at[scat[p]], sem_w[p], add=True)

    start_seg(0)
    wait_seg(0)
    start_gather(tbig[0], 0, 0)

    for seg in range(NSEG):
        sb, tb = sbig[seg % 2], tbig[seg % 2]
        nxt = seg + 1 < NSEG
        tbn = tbig[(seg + 1) % 2] if nxt else None
        if nxt:
            start_seg(seg + 1)
        par = (SEG * seg) % 2

        if seg == 0:
            step(0, 1, sb, tb, 0, tb, 1, first_r=True, first_w=True)
            step(1, 0, sb, tb, 1, tb, 2, first_w=True)
            body_lo, body_pairs = 2, (SEG - 1 - 2) // 2
        else:
            body_lo, body_pairs = 0, (SEG - 1) // 2

        @pl.loop(0, body_pairs)
        def _pairs(j):
            c = body_lo + 2 * j
            step(par, 1 - par, sb, tb, c, tb, c + 1)
            step(1 - par, par, sb, tb, c + 1, tb, c + 2)

        lpar = (SEG * seg + SEG - 1) % 2
        if nxt:
            wait_seg(seg + 1)
            step(lpar, 1 - lpar, sb, tb, SEG - 1, tbn, 0)
        else:
            step(lpar, 1 - lpar, sb, tb, SEG - 1)

    pltpu.make_async_copy(rows[1], aggsh.at[scat[1]], sem_r[1]).wait()
    pltpu.make_async_copy(wbuf[1], divsh.at[scat[1]], sem_w[1]).wait()
    pltpu.make_async_copy(rows[0], aggsh.at[scat[0]], sem_r[0]).wait()
    pltpu.make_async_copy(wbuf[0], divsh.at[scat[0]], sem_w[0]).wait()

    plsc.subcore_barrier()

    for k in range(KZ):
        blk = sid + NS * k

        @pl.when(blk < NZB)
        def _writeback():
            r0 = pl.multiple_of(blk * ZBLK, ZBLK)
            pltpu.sync_copy(aggsh.at[pl.ds(r0, ZBLK)],
                            agg_hbm.at[cid, pl.ds(r0, ZBLK)])

    @pl.when(sid == 0)
    def _writeback_div():
        pltpu.sync_copy(divsh, div_hbm.at[cid])

  return _sc_edge


def _combine_body(a0_ref, a1_ref, d0_ref, d1_ref, o_ref):
    o_ref[...] = (a0_ref[...] + a1_ref[...]) / (d0_ref[...] + d1_ref[...])


def _combine(a0, a1, d0, d1):
    B = 2000
    return pl.pallas_call(
        _combine_body,
        grid=(N // B,),
        in_specs=[
            pl.BlockSpec((B, D), lambda i: (i, 0)),
            pl.BlockSpec((B, D), lambda i: (i, 0)),
            pl.BlockSpec((B, 1), lambda i: (i, 0)),
            pl.BlockSpec((B, 1), lambda i: (i, 0)),
        ],
        out_specs=pl.BlockSpec((B, D), lambda i: (i, 0)),
        out_shape=jax.ShapeDtypeStruct((N, D), jnp.float32),
    )(a0, a1, d0, d1)


def kernel(x, s, t, W_lin, b_lin, W_attn):
    a_mat = W_attn.reshape(2, D)
    h, fg = _dense(x, W_lin, b_lin.reshape(1, D), a_mat)
    aggs, divs = _make_sc_edge()(h, fg.reshape(2 * N), s, t)
    return _combine(aggs[0], aggs[1],
                    divs[0].reshape(N, 1), divs[1].reshape(N, 1))


# speedup vs baseline: 16.3726x; 1.0142x over previous
import functools

import jax
import jax.numpy as jnp
from jax import lax
from jax.experimental import pallas as pl
from jax.experimental.pallas import tpu as pltpu
from jax.experimental.pallas import tpu_sc as plsc

N = 10000
E = 320000
D = 128
ALPHA = 0.2

NC, NS = 2, 16
NW = NC * NS
C = 80
CPW = E // C // NW
SEG = 25
NSEG = CPW // SEG
ZBLK = 80
NZB = N // ZBLK
KZ = -(-NZB // NS)


def _dense_body(x_ref, w_ref, b_ref, a_ref, h_ref, fg_ref):
    h = lax.dot_general(x_ref[...], w_ref[...], (((1,), (1,)), ((), ())),
                        preferred_element_type=jnp.float32) + b_ref[...]
    h_ref[...] = h
    fg_ref[...] = lax.dot_general(a_ref[...], h, (((1,), (1,)), ((), ())),
                                  preferred_element_type=jnp.float32)


def _dense(x, W_lin, b_lin, a_mat):
    return pl.pallas_call(
        _dense_body,
        out_shape=[
            jax.ShapeDtypeStruct((N, D), jnp.float32),
            jax.ShapeDtypeStruct((2, N), jnp.float32),
        ],
    )(x, W_lin, b_lin, a_mat)


@functools.cache
def _make_sc_edge():
  mesh = plsc.VectorSubcoreMesh(core_axis_name="c", subcore_axis_name="s")

  @functools.partial(
      pl.kernel,
      mesh=mesh,
      compiler_params=pltpu.CompilerParams(
          needs_layout_passes=False, use_tc_tiling_on_sc=False),
      out_type=[
          jax.ShapeDtypeStruct((NC, N, D), jnp.float32),
          jax.ShapeDtypeStruct((NC, N), jnp.float32),
      ],
      scratch_types=[
          pltpu.VMEM((SEG * C,), jnp.int32),
          pltpu.VMEM((SEG * C,), jnp.int32),
          pltpu.VMEM((SEG * C,), jnp.int32),
          pltpu.VMEM((SEG * C,), jnp.int32),
          pltpu.VMEM((C,), jnp.int32),
          pltpu.VMEM((C,), jnp.int32),
          pltpu.VMEM((C, D), jnp.float32),
          pltpu.VMEM((C, D), jnp.float32),
          pltpu.VMEM((C,), jnp.float32),
          pltpu.VMEM((C,), jnp.float32),
          pltpu.VMEM((N,), jnp.float32),
          pltpu.VMEM((N,), jnp.float32),
          pltpu.VMEM_SHARED((N, D), jnp.float32),
          pltpu.VMEM_SHARED((N,), jnp.float32),
          pltpu.SemaphoreType.DMA,
          pltpu.SemaphoreType.DMA,
          pltpu.SemaphoreType.DMA,
          pltpu.SemaphoreType.DMA,
          pltpu.SemaphoreType.DMA,
          pltpu.SemaphoreType.DMA,
          pltpu.SemaphoreType.DMA,
          pltpu.SemaphoreType.DMA,
          pltpu.SemaphoreType.DMA,
          pltpu.SemaphoreType.DMA,
      ],
  )
  def _sc_edge(h_hbm, fg_hbm, s_hbm, t_hbm, agg_hbm, div_hbm,
               sbig0, sbig1, tbig0, tbig1, scat0, scat1, rows0, rows1,
               wbuf0, wbuf1, ftab, gtab, aggsh, divsh,
               ss0, ss1, st0, st1, sg0, sg1, sr0, sr1, sw0, sw1):
    cid = lax.axis_index("c")
    sid = lax.axis_index("s")
    wid = cid * NS + sid
    base = wid * CPW * C

    sbig = (sbig0, sbig1)
    tbig = (tbig0, tbig1)
    scat = (scat0, scat1)
    rows = (rows0, rows1)
    wbuf = (wbuf0, wbuf1)
    sem_s = (ss0, ss1)
    sem_t = (st0, st1)
    sem_g = (sg0, sg1)
    sem_r = (sr0, sr1)
    sem_w = (sw0, sw1)

    zeros16 = jnp.zeros((16,), jnp.float32)

    @pl.loop(0, N // 16)
    def _zero_ftab(i):
        ftab[pl.ds(pl.multiple_of(i * 16, 16), 16)] = zeros16

    @pl.when(sid == 0)
    def _zero_div():
        pltpu.sync_copy(ftab, divsh)

    @pl.loop(0, C)
    def _zero_rows0(i):
        for j in range(D // 16):
            rows0[i, pl.ds(j * 16, 16)] = zeros16

    for k in range(KZ):
        blk = sid + NS * k

        @pl.when(blk < NZB)
        def _zero_agg():
            pltpu.sync_copy(
                rows0, aggsh.at[pl.ds(pl.multiple_of(blk * ZBLK, ZBLK), ZBLK)])

    pltpu.sync_copy(fg_hbm.at[pl.ds(0, N)], ftab)
    pltpu.sync_copy(fg_hbm.at[pl.ds(N, N)], gtab)

    plsc.subcore_barrier()

    def start_seg(g):
        eb = pl.multiple_of(base + g * SEG * C, C)
        m = g % 2
        pltpu.async_copy(s_hbm.at[pl.ds(eb, SEG * C)], sbig[m], sem_s[m])
        pltpu.async_copy(t_hbm.at[pl.ds(eb, SEG * C)], tbig[m], sem_t[m])

    def wait_seg(g):
        eb = pl.multiple_of(base + g * SEG * C, C)
        m = g % 2
        pltpu.make_async_copy(
            s_hbm.at[pl.ds(eb, SEG * C)], sbig[m], sem_s[m]).wait()
        pltpu.make_async_copy(
            t_hbm.at[pl.ds(eb, SEG * C)], tbig[m], sem_t[m]).wait()

    H = C // 2

    def gidx(tb, lc, half):
        return tb.at[pl.ds(pl.multiple_of(lc * C + half * H, H), H)]

    def start_gather(tb, lc, q):
        pltpu.async_copy(
            h_hbm.at[gidx(tb, lc, 0)], rows[q].at[pl.ds(0, H)], sem_g[q])
        pltpu.async_copy(
            h_hbm.at[gidx(tb, lc, 1)], rows[q].at[pl.ds(H, H)], sem_g[q])

    def step(p, q, sb, tb, lc, tb1=None, lc1=None,
             first_r=False, first_w=False):
        pltpu.make_async_copy(
            h_hbm.at[gidx(tb, lc, 0)], rows[p].at[pl.ds(0, H)], sem_g[p]).wait()
        pltpu.make_async_copy(
            h_hbm.at[gidx(tb, lc, 1)], rows[p].at[pl.ds(H, H)], sem_g[p]).wait()

        if tb1 is not None:
            if not first_r:
                pltpu.make_async_copy(
                    rows[q], aggsh.at[scat[q]], sem_r[q]).wait()
            start_gather(tb1, lc1, q)

        if not first_w:
            pltpu.make_async_copy(
                wbuf[p], divsh.at[scat[p]], sem_w[p]).wait()
        cbase = lc * C
        for grp in range(C // 16):
            off = pl.multiple_of(cbase + grp * 16, 16)
            sv = sb[pl.ds(off, 16)]
            tv = tb[pl.ds(off, 16)]
            e = plsc.load_gather(ftab, [sv]) + plsc.load_gather(gtab, [tv])
            e = jnp.where(e >= 0.0, e, ALPHA * e)
            wbuf[p][pl.ds(grp * 16, 16)] = jnp.exp(e)
            scat[p][pl.ds(grp * 16, 16)] = sv

        @plsc.parallel_loop(0, C, unroll=4)
        def _scale(i):
            wv = plsc.load_gather(wbuf[p], [jnp.full((16,), i, jnp.int32)])
            for j in range(D // 16):
                rows[p][i, pl.ds(j * 16, 16)] = (
                    rows[p][i, pl.ds(j * 16, 16)] * wv)

        pltpu.async_copy(rows[p], aggsh.at[scat[p]], sem_r[p], add=True)
        pltpu.async_copy(wbuf[p], divsh.at[scat[p]], sem_w[p], add=True)

    start_seg(0)
    wait_seg(0)
    start_gather(tbig[0], 0, 0)

    for seg in range(NSEG):
        sb, tb = sbig[seg % 2], tbig[seg % 2]
        nxt = seg + 1 < NSEG
        tbn = tbig[(seg + 1) % 2] if nxt else None
        if nxt:
            start_seg(seg + 1)
        par = (SEG * seg) % 2

        if seg == 0:
            step(0, 1, sb, tb, 0, tb, 1, first_r=True, first_w=True)
            step(1, 0, sb, tb, 1, tb, 2, first_w=True)
            body_lo, body_pairs = 2, (SEG - 1 - 2) // 2
        else:
            body_lo, body_pairs = 0, (SEG - 1) // 2

        @pl.loop(0, body_pairs)
        def _pairs(j):
            c = body_lo + 2 * j
            step(par, 1 - par, sb, tb, c, tb, c + 1)
            step(1 - par, par, sb, tb, c + 1, tb, c + 2)

        lpar = (SEG * seg + SEG - 1) % 2
        if nxt:
            wait_seg(seg + 1)
            step(lpar, 1 - lpar, sb, tb, SEG - 1, tbn, 0)
        else:
            step(lpar, 1 - lpar, sb, tb, SEG - 1)

    pltpu.make_async_copy(rows[1], aggsh.at[scat[1]], sem_r[1]).wait()
    pltpu.make_async_copy(wbuf[1], divsh.at[scat[1]], sem_w[1]).wait()
    pltpu.make_async_copy(rows[0], aggsh.at[scat[0]], sem_r[0]).wait()
    pltpu.make_async_copy(wbuf[0], divsh.at[scat[0]], sem_w[0]).wait()

    plsc.subcore_barrier()

    for k in range(KZ):
        blk = sid + NS * k

        @pl.when(blk < NZB)
        def _writeback():
            r0 = pl.multiple_of(blk * ZBLK, ZBLK)
            pltpu.sync_copy(aggsh.at[pl.ds(r0, ZBLK)],
                            agg_hbm.at[cid, pl.ds(r0, ZBLK)])

    @pl.when(sid == 0)
    def _writeback_div():
        pltpu.sync_copy(divsh, div_hbm.at[cid])

  return _sc_edge


def _combine_body(a0_ref, a1_ref, d0_ref, d1_ref, o_ref):
    o_ref[...] = (a0_ref[...] + a1_ref[...]) / (d0_ref[...] + d1_ref[...])


def _combine(a0, a1, d0, d1):
    B = 2000
    return pl.pallas_call(
        _combine_body,
        grid=(N // B,),
        in_specs=[
            pl.BlockSpec((B, D), lambda i: (i, 0)),
            pl.BlockSpec((B, D), lambda i: (i, 0)),
            pl.BlockSpec((B, 1), lambda i: (i, 0)),
            pl.BlockSpec((B, 1), lambda i: (i, 0)),
        ],
        out_specs=pl.BlockSpec((B, D), lambda i: (i, 0)),
        out_shape=jax.ShapeDtypeStruct((N, D), jnp.float32),
    )(a0, a1, d0, d1)


def kernel(x, s, t, W_lin, b_lin, W_attn):
    a_mat = W_attn.reshape(2, D)
    h, fg = _dense(x, W_lin, b_lin.reshape(1, D), a_mat)
    aggs, divs = _make_sc_edge()(h, fg.reshape(2 * N), s, t)
    return _combine(aggs[0], aggs[1],
                    divs[0].reshape(N, 1), divs[1].reshape(N, 1))
